# baseline (speedup 1.0000x reference)
import functools

import jax
import jax.numpy as jnp
from jax import lax
from jax.experimental import pallas as pl
from jax.experimental.pallas import tpu as pltpu
from jax.experimental.pallas import tpu_sc as plsc

NC = 2
NS = 16
NW = NC * NS
CH = 128
K = 8


@functools.partial(jax.jit, static_argnames=("n_ch",))
def _embed_sc(idx3, weight, n_ch):
    n = NW * n_ch * CH
    d = weight.shape[1]
    per_w = n_ch * CH

    mesh = plsc.VectorSubcoreMesh(core_axis_name="c", subcore_axis_name="s")

    @functools.partial(
        pl.kernel,
        out_type=jax.ShapeDtypeStruct((n, d), jnp.float32),
        mesh=mesh,
        compiler_params=pltpu.CompilerParams(use_tc_tiling_on_sc=False),
        scratch_types=[
            pltpu.VMEM((n_ch, CH), jnp.int32),
            *([pltpu.VMEM((CH, d), jnp.float32)] * K),
            *([pltpu.SemaphoreType.DMA] * K),
            *([pltpu.SemaphoreType.DMA] * K),
        ],
    )
    def body(idx_hbm, table_hbm, out_hbm, idx_v, *rest):
        bufs = rest[:K]
        gsem = rest[K : 2 * K]
        ssem = rest[2 * K : 3 * K]
        wid = lax.axis_index("s") * NC + lax.axis_index("c")
        base = wid * per_w
        pltpu.sync_copy(idx_hbm.at[wid], idx_v)

        def block(b, _):
            gathers = []
            for k in range(K):
                j = b * K + k
                gathers.append(
                    pltpu.async_copy(table_hbm.at[idx_v.at[j]], bufs[k], gsem[k])
                )
            scatters = []
            for k in range(K):
                j = b * K + k
                gathers[k].wait()
                scatters.append(
                    pltpu.async_copy(
                        bufs[k], out_hbm.at[pl.ds(base + j * CH, CH)], ssem[k]
                    )
                )
            for k in range(K):
                scatters[k].wait()
            return _

        lax.fori_loop(0, n_ch // K, block, None)

    return body(idx3, weight)


def kernel(x, weight):
    b, h = x.shape
    d = weight.shape[1]
    n = b * h
    n_ch = n // (NW * CH)
    idx3 = x.reshape(NW, n_ch, CH).astype(jnp.int32)
    out = _embed_sc(idx3, weight, n_ch)
    return out.reshape(b, h, d)
